# baseline (speedup 1.0000x reference)
import functools

import jax
import jax.numpy as jnp
from jax import lax
from jax.experimental import pallas as pl
from jax.experimental.pallas import tpu as pltpu
from jax.experimental.pallas import tpu_sc as plsc

_N = 10000
_D = 128
_H = 4
_DK = 32
_E = 320000
_DFF = 512

_NC = 2
_NS = 16
_NW = _NC * _NS
_L = 16

_RB = 1000
_C = 80
_EPT = _E // _NW
_EPC = _E // _NS
_RPT = _N // _NS
_ISD = 1.0 / (_DK ** 0.5)

_f32 = jnp.float32
_i32 = jnp.int32


def _mm(a, b):
    return lax.dot_general(a, b, (((1,), (0,)), ((), ())),
                           preferred_element_type=_f32)


def _t1_body(mu_ref, var_ref, hm_ref,
             wqm, bqm, wqv, bqv, wkm, bkm, wkv, bkv, wvm, bvm, wvv, bvv,
             x_ref, y_ref, cd_ref, cs_ref, vm_ref, vv_ref):
    mu = mu_ref[...]
    var = var_ref[...]
    sp = jax.nn.softplus
    qm = _mm(mu, wqm[...]) + bqm[...]
    qv = sp(_mm(var, wqv[...]) + bqv[...])
    km = _mm(mu, wkm[...]) + bkm[...]
    kv = sp(_mm(var, wkv[...]) + bkv[...])
    vm_ref[...] = _mm(mu, wvm[...]) + bvm[...]
    vv_ref[...] = sp(_mm(var, wvv[...]) + bvv[...])
    ikv = 1.0 / kv
    x_ref[:, 0:_D] = qm * qm + qv
    x_ref[:, _D:2 * _D] = qm
    y_ref[:, 0:_D] = 0.5 * ikv
    y_ref[:, _D:2 * _D] = -(km * ikv)
    hm = hm_ref[...]
    cd_ref[...] = _mm(-0.5 * jnp.log(qv), hm)
    cs_ref[...] = _mm(0.5 * km * km * ikv + 0.5 * jnp.log(kv), hm) - (_DK / 2.0)


def _t1(mu, var, hmask, p):
    nb = _N // _RB
    row = lambda i: (i, 0)
    full = lambda i: (0, 0)
    rspec = pl.BlockSpec((_RB, _D), row)
    wspec = pl.BlockSpec((_D, _D), full)
    bspec = pl.BlockSpec((1, _D), full)
    outs = (
        jax.ShapeDtypeStruct((_N, 2 * _D), _f32),
        jax.ShapeDtypeStruct((_N, 2 * _D), _f32),
        jax.ShapeDtypeStruct((_N, _H), _f32),
        jax.ShapeDtypeStruct((_N, _H), _f32),
        jax.ShapeDtypeStruct((_N, _D), _f32),
        jax.ShapeDtypeStruct((_N, _D), _f32),
    )
    out_specs = (
        pl.BlockSpec((_RB, 2 * _D), row),
        pl.BlockSpec((_RB, 2 * _D), row),
        pl.BlockSpec((_RB, _H), row),
        pl.BlockSpec((_RB, _H), row),
        rspec,
        rspec,
    )
    return pl.pallas_call(
        _t1_body,
        grid=(nb,),
        in_specs=[rspec, rspec, pl.BlockSpec((_D, _H), full)] +
                 [wspec, bspec] * 6,
        out_specs=out_specs,
        out_shape=outs,
    )(mu, var, hmask,
      p['W_qm'], p['b_qm'], p['W_qv'], p['b_qv'],
      p['W_km'], p['b_km'], p['W_kv'], p['b_kv'],
      p['W_vm'], p['b_vm'], p['W_vv'], p['b_vv'])


def _s1_body(x_hbm, y_hbm, cd_hbm, cs_hbm, dst_hbm, src_hbm,
             scores_hbm,
             idxd, idxs, xr, yr, cdv, csv, so, sem):
    c = lax.axis_index("c")
    s = lax.axis_index("s")
    wid = s * _NC + c
    ebase = wid * _EPT
    lanes = lax.broadcasted_iota(_i32, (_L,), 0)

    def batch(i, _):
        b0 = ebase + i * _C
        pltpu.sync_copy(dst_hbm.at[pl.ds(b0, _C)], idxd)
        pltpu.sync_copy(src_hbm.at[pl.ds(b0, _C)], idxs)
        cp1 = pltpu.async_copy(x_hbm.at[idxd], xr, sem)
        cp2 = pltpu.async_copy(y_hbm.at[idxs], yr, sem)
        cp3 = pltpu.async_copy(cd_hbm.at[idxd], cdv, sem)
        cp4 = pltpu.async_copy(cs_hbm.at[idxs], csv, sem)
        cp1.wait(); cp2.wait(); cp3.wait(); cp4.wait()

        def group(g, _):
            rows = g * _L + lanes
            acc = [jnp.zeros((_L,), _f32) for _ in range(_H)]
            for d in range(2 * _D):
                col = jnp.full((_L,), d, _i32)
                xv = plsc.load_gather(xr, [rows, col])
                yv = plsc.load_gather(yr, [rows, col])
                h = (d // _DK) % _H
                acc[h] = acc[h] + xv * yv
            for h in range(_H):
                hcol = jnp.full((_L,), h, _i32)
                cdh = plsc.load_gather(cdv, [rows, hcol])
                csh = plsc.load_gather(csv, [rows, hcol])
                sc = -(acc[h] + cdh + csh) * _ISD
                plsc.store_scatter(so, [rows, hcol], sc)
            return 0

        lax.fori_loop(0, _C // _L, group, 0)
        pltpu.sync_copy(so, scores_hbm.at[pl.ds(b0, _C)])
        return 0

    lax.fori_loop(0, _EPT // _C, batch, 0)


def _s1(x, y, cdst, csrc, dst, src):
    mesh = plsc.VectorSubcoreMesh(core_axis_name="c", subcore_axis_name="s",
                                  num_cores=_NC, num_subcores=_NS)
    fn = pl.kernel(
        _s1_body,
        out_type=jax.ShapeDtypeStruct((_E, _H), _f32),
        mesh=mesh,
        compiler_params=pltpu.CompilerParams(use_tc_tiling_on_sc=False, needs_layout_passes=False),
        scratch_types=[
            pltpu.VMEM((_C,), _i32),
            pltpu.VMEM((_C,), _i32),
            pltpu.VMEM((_C, 2 * _D), _f32),
            pltpu.VMEM((_C, 2 * _D), _f32),
            pltpu.VMEM((_C, _H), _f32),
            pltpu.VMEM((_C, _H), _f32),
            pltpu.VMEM((_C, _H), _f32),
            pltpu.SemaphoreType.DMA,
        ],
    )
    return fn(x, y, cdst, csrc, dst, src)


_SR = 2000
_SC_COLS = 640
_SB = 200

def _t2_body(s_ref, p_ref, z_ref, msc, zsc):
    ph = pl.program_id(0)
    j = pl.program_id(1)
    nb = pl.num_programs(1)
    lane = lax.broadcasted_iota(_i32, (_SB, _SC_COLS), 1) % _H
    s = s_ref[...]

    @pl.when(jnp.logical_and(ph == 0, j == 0))
    def _init_m():
        for h in range(_H):
            msc[h] = _f32(-1e30)

    @pl.when(ph == 0)
    def _maxpass():
        for h in range(_H):
            mh = jnp.max(jnp.where(lane == h, s, _f32(-1e30)))
            msc[h] = jnp.maximum(msc[h], mh)

    @pl.when(jnp.logical_and(ph == 1, j == 0))
    def _init_z():
        for h in range(_H):
            zsc[h] = _f32(0.0)

    @pl.when(ph == 1)
    def _exppass():
        mvec = jnp.zeros((_SB, _SC_COLS), _f32)
        for h in range(_H):
            mvec = mvec + jnp.where(lane == h, msc[h], _f32(0.0))
        pv = jnp.exp(s - mvec)
        p_ref[...] = pv
        for h in range(_H):
            zsc[h] = zsc[h] + jnp.sum(jnp.where(lane == h, pv, _f32(0.0)))

    @pl.when(jnp.logical_and(ph == 1, j == nb - 1))
    def _emit_z():
        i4 = lax.broadcasted_iota(_i32, (1, _H), 1)
        zv = jnp.zeros((1, _H), _f32)
        for h in range(_H):
            zv = zv + jnp.where(i4 == h, zsc[h], _f32(0.0))
        z_ref[...] = zv


def _t2(scores2d):
    nb = _SR // _SB
    return pl.pallas_call(
        _t2_body,
        grid=(2, nb),
        in_specs=[pl.BlockSpec((_SB, _SC_COLS), lambda p, j: (j, 0))],
        out_specs=(pl.BlockSpec((_SB, _SC_COLS), lambda p, j: (j, 0)),
                   pl.BlockSpec((1, _H), lambda p, j: (0, 0))),
        out_shape=(jax.ShapeDtypeStruct((_SR, _SC_COLS), _f32),
                   jax.ShapeDtypeStruct((1, _H), _f32)),
        scratch_shapes=[pltpu.SMEM((_H,), _f32), pltpu.SMEM((_H,), _f32)],
    )(scores2d)


def _s2_body(vm_hbm, vv_hbm, p_hbm, dst_hbm, src_hbm,
             mau_hbm, vau_hbm,
             idxd, idxs, pv, rows_v, agg, sem):
    c = lax.axis_index("c")
    s = lax.axis_index("s")
    lanes = lax.broadcasted_iota(_i32, (_L,), 0)
    zeros = jnp.zeros((_L,), _f32)

    def zrow(r, _):
        for k in range(_D // _L):
            rows_v[r, pl.ds(k * _L, _L)] = zeros
        return 0
    lax.fori_loop(0, _C, zrow, 0)
    for k in range(7):
        pltpu.sync_copy(rows_v, agg.at[pl.ds(s * _RPT + k * _C, _C)])
    pltpu.sync_copy(rows_v.at[pl.ds(0, _RPT - 7 * _C)],
                    agg.at[pl.ds(s * _RPT + 7 * _C, _RPT - 7 * _C)])
    plsc.subcore_barrier()

    def run(table_hbm, out_hbm, square):
        def batch(i, _):
            b0 = s * _EPC + i * _C
            pltpu.sync_copy(dst_hbm.at[pl.ds(b0, _C)], idxd)
            pltpu.sync_copy(src_hbm.at[pl.ds(b0, _C)], idxs)
            cp1 = pltpu.async_copy(table_hbm.at[idxs], rows_v, sem)
            pltpu.sync_copy(p_hbm.at[pl.ds(b0, _C)], pv)
            cp1.wait()

            def group(g, _):
                rows = g * _L + lanes
                w4 = []
                for h in range(_H):
                    wv = plsc.load_gather(pv, [rows, jnp.full((_L,), h, _i32)])
                    if square:
                        wv = wv * wv
                    w4.append(wv)
                for d in range(_D):
                    col = jnp.full((_L,), d, _i32)
                    v = plsc.load_gather(rows_v, [rows, col])
                    plsc.store_scatter(rows_v, [rows, col], v * w4[d // _DK])
                return 0

            lax.fori_loop(0, _C // _L, group, 0)
            pltpu.sync_copy(rows_v, agg.at[idxd], add=True)
            return 0

        lax.fori_loop(0, _EPC // _C, batch, 0)
        plsc.subcore_barrier()
        for k in range(7):
            pltpu.sync_copy(agg.at[pl.ds(s * _RPT + k * _C, _C)], rows_v)
            pltpu.sync_copy(rows_v, out_hbm.at[pl.ds(s * _RPT + k * _C, _C)])
        tail = _RPT - 7 * _C
        pltpu.sync_copy(agg.at[pl.ds(s * _RPT + 7 * _C, tail)],
                        rows_v.at[pl.ds(0, tail)])
        pltpu.sync_copy(rows_v.at[pl.ds(0, tail)],
                        out_hbm.at[pl.ds(s * _RPT + 7 * _C, tail)])

    @pl.when(c == 0)
    def _mu_side():
        run(vm_hbm, mau_hbm, False)

    @pl.when(c == 1)
    def _var_side():
        run(vv_hbm, vau_hbm, True)


def _s2(vm, vv, pw, dst, src):
    mesh = plsc.VectorSubcoreMesh(core_axis_name="c", subcore_axis_name="s",
                                  num_cores=_NC, num_subcores=_NS)
    fn = pl.kernel(
        _s2_body,
        out_type=(jax.ShapeDtypeStruct((_N, _D), _f32),
                  jax.ShapeDtypeStruct((_N, _D), _f32)),
        mesh=mesh,
        compiler_params=pltpu.CompilerParams(use_tc_tiling_on_sc=False, needs_layout_passes=False),
        scratch_types=[
            pltpu.VMEM((_C,), _i32),
            pltpu.VMEM((_C,), _i32),
            pltpu.VMEM((_C, _H), _f32),
            pltpu.VMEM((_C, _D), _f32),
            pltpu.VMEM_SHARED((_N, _D), _f32),
            pltpu.SemaphoreType.DMA,
        ],
    )
    return fn(vm, vv, pw, dst, src)


def _ln(x, g, b):
    m = jnp.mean(x, axis=-1, keepdims=True)
    v = jnp.mean((x - m) ** 2, axis=-1, keepdims=True)
    return (x - m) * lax.rsqrt(v + 1e-5) * g + b


def _t3_body(mu_ref, var_ref, mau_ref, vau_ref, z_ref, hm_ref,
             wom, bom, wov, bov, g1, be1, w1, b1, w2, b2, g2, be2,
             mu2_ref, var2_ref):
    sp = jax.nn.softplus
    invz = 1.0 / z_ref[...]
    colscale = lax.dot_general(invz, hm_ref[...],
                               (((1,), (1,)), ((), ())),
                               preferred_element_type=_f32)
    mu_agg = mau_ref[...] * colscale
    var_agg = vau_ref[...] * (colscale * colscale)
    attn_mu = _mm(mu_agg, wom[...]) + bom[...]
    attn_var = sp(_mm(var_agg, wov[...]) + bov[...])
    mu1 = _ln(mu_ref[...] + attn_mu, g1[...], be1[...])
    var1 = _ln(var_ref[...] + attn_var, g1[...], be1[...])
    hx = _mm(mu1, w1[...]) + b1[...]
    h = hx * (0.5 * (1.0 + lax.erf(hx * (2.0 ** -0.5))))
    ff = _mm(h, w2[...]) + b2[...]
    mu2_ref[...] = _ln(mu1 + ff, g2[...], be2[...])
    var2_ref[...] = _ln(var1 + var1, g2[...], be2[...])


def _t3(mu, var, mau, vau, z, hmask, p):
    nb = _N // _RB
    row = lambda i: (i, 0)
    full = lambda i: (0, 0)
    rspec = pl.BlockSpec((_RB, _D), row)
    bspec = pl.BlockSpec((1, _D), full)
    return pl.pallas_call(
        _t3_body,
        grid=(nb,),
        in_specs=[rspec, rspec, rspec, rspec,
                  pl.BlockSpec((1, _H), full), pl.BlockSpec((_D, _H), full),
                  pl.BlockSpec((_D, _D), full), bspec,
                  pl.BlockSpec((_D, _D), full), bspec,
                  bspec, bspec,
                  pl.BlockSpec((_D, _DFF), full), pl.BlockSpec((1, _DFF), full),
                  pl.BlockSpec((_DFF, _D), full), bspec,
                  bspec, bspec],
        out_specs=(rspec, rspec),
        out_shape=(jax.ShapeDtypeStruct((_N, _D), _f32),
                   jax.ShapeDtypeStruct((_N, _D), _f32)),
    )(mu, var, mau, vau, z, hmask,
      p['W_om'], p['b_om'], p['W_ov'], p['b_ov'],
      p['g1'], p['be1'], p['W1'], p['b1'], p['W2'], p['b2'],
      p['g2'], p['be2'])


def kernel(mu, var, edge_index, params):
    p = {k: (v.reshape(1, -1) if v.ndim == 1 else v)
         for k, v in params.items()}
    src = edge_index[0]
    dst = edge_index[1]
    hmask = (lax.broadcasted_iota(_i32, (_D, _H), 0) // _DK ==
             lax.broadcasted_iota(_i32, (_D, _H), 1)).astype(_f32)

    x, y, cdst, csrc, vm, vv = _t1(mu, var, hmask, p)
    scores = _s1(x, y, cdst, csrc, dst, src)
    pw2d, z = _t2(scores.reshape(_SR, _SC_COLS))
    pw = pw2d.reshape(_E, _H)
    mau, vau = _s2(vm, vv, pw, dst, src)
    return _t3(mu, var, mau, vau, z, hmask, p)
